# baseline (speedup 1.0000x reference)
import functools

import jax
import jax.numpy as jnp
from jax import lax
from jax.experimental import pallas as pl
from jax.experimental.pallas import tpu as pltpu
from jax.experimental.pallas import tpu_sc as plsc

N, T, H, E, C, K, TP, L = 2048, 16, 128, 32768, 128, 5, 64, 2

BI = 256
BK = 512
NI = N // BI
NK = N // BK
BN = 256
NB = N // BN


def _sage_body(aff_ref, aee_ref, afe_ref, xfs_ref, xes_ref, xfd_ref, xed_ref,
               wrf_ref, wnf_ref, bf_ref, wre_ref, wnee_ref, wnfe_ref, be_ref,
               of_ref, oe_ref,
               accf, acce, accfe, cntf, cnte, cntfe):
    k = pl.program_id(1)

    @pl.when(k == 0)
    def _init():
        accf[...] = jnp.zeros_like(accf)
        acce[...] = jnp.zeros_like(acce)
        accfe[...] = jnp.zeros_like(accfe)
        cntf[...] = jnp.zeros_like(cntf)
        cnte[...] = jnp.zeros_like(cnte)
        cntfe[...] = jnp.zeros_like(cntfe)

    aff = aff_ref[...]
    aee = aee_ref[...]
    afe = afe_ref[...]
    cntf[...] += jnp.sum(aff.astype(jnp.float32), axis=1, keepdims=True)
    cnte[...] += jnp.sum(aee.astype(jnp.float32), axis=1, keepdims=True)
    cntfe[...] += jnp.sum(afe.astype(jnp.float32), axis=1, keepdims=True)
    for t in range(T):
        xf_t = xfs_ref[:, t, :]
        xe_t = xes_ref[:, t, :]
        accf[:, t, :] += jnp.dot(aff, xf_t, preferred_element_type=jnp.float32)
        acce[:, t, :] += jnp.dot(aee, xe_t, preferred_element_type=jnp.float32)
        accfe[:, t, :] += jnp.dot(afe, xf_t, preferred_element_type=jnp.float32)

    @pl.when(k == NK - 1)
    def _fin():
        invf = 1.0 / jnp.maximum(cntf[...], 1.0)
        inve = 1.0 / jnp.maximum(cnte[...], 1.0)
        invfe = 1.0 / jnp.maximum(cntfe[...], 1.0)
        aggf = (accf[...] * invf[:, :, None]).reshape(BI * T, H)
        agge = (acce[...] * inve[:, :, None]).reshape(BI * T, H)
        aggfe = (accfe[...] * invfe[:, :, None]).reshape(BI * T, H)
        xdf = xfd_ref[...].reshape(BI * T, H)
        xde = xed_ref[...].reshape(BI * T, H)
        of = (jnp.dot(xdf, wrf_ref[...], preferred_element_type=jnp.float32)
              + jnp.dot(aggf, wnf_ref[...], preferred_element_type=jnp.float32)
              + bf_ref[...])
        oe = (jnp.dot(xde, wre_ref[...], preferred_element_type=jnp.float32)
              + jnp.dot(agge, wnee_ref[...], preferred_element_type=jnp.float32)
              + jnp.dot(aggfe, wnfe_ref[...], preferred_element_type=jnp.float32)
              + be_ref[...])
        of_ref[...] = jnp.maximum(of, 0.0).reshape(BI, T, H)
        oe_ref[...] = jnp.maximum(0.5 * oe, 0.0).reshape(BI, T, H)


def _sage_layer(a_ff, a_ee, a_fe, xfb, xeb, xf, xe,
                wrf, wnf, bf, wre, wnee, wnfe, be):
    grid = (NI, NK)
    adj_spec = pl.BlockSpec((BI, BK), lambda i, k: (i, k))
    src_spec = pl.BlockSpec((BK, T, H), lambda i, k: (k, 0, 0))
    dst_spec = pl.BlockSpec((BI, T, H), lambda i, k: (i, 0, 0))
    w_spec = pl.BlockSpec((H, H), lambda i, k: (0, 0))
    b_spec = pl.BlockSpec((1, H), lambda i, k: (0, 0))
    out_spec = pl.BlockSpec((BI, T, H), lambda i, k: (i, 0, 0))
    return pl.pallas_call(
        _sage_body,
        grid=grid,
        in_specs=[adj_spec, adj_spec, adj_spec, src_spec, src_spec,
                  dst_spec, dst_spec,
                  w_spec, w_spec, b_spec, w_spec, w_spec, w_spec, b_spec],
        out_specs=[out_spec, out_spec],
        out_shape=[jax.ShapeDtypeStruct((N, T, H), jnp.float32)] * 2,
        scratch_shapes=[pltpu.VMEM((BI, T, H), jnp.float32)] * 3
        + [pltpu.VMEM((BI, 1), jnp.float32)] * 3,
        compiler_params=pltpu.CompilerParams(
            dimension_semantics=("parallel", "arbitrary")),
    )(a_ff, a_ee, a_fe, xfb, xeb, xf, xe, wrf, wnf, bf, wre, wnee, wnfe, be)


def _gru_body(x_ref, wih_ref, whh_ref, bih_ref, bhh_ref, ys_ref, hf_ref):
    x = x_ref[...].reshape(BN * T, H)
    gi = (jnp.dot(x, wih_ref[...], preferred_element_type=jnp.float32)
          + bih_ref[...]).reshape(BN, T, 3 * H)
    h = jnp.zeros((BN, H), jnp.float32)
    for t in range(T):
        gi_t = gi[:, t, :]
        gh = jnp.dot(h, whh_ref[...], preferred_element_type=jnp.float32) \
            + bhh_ref[...]
        r = jax.nn.sigmoid(gi_t[:, :H] + gh[:, :H])
        z = jax.nn.sigmoid(gi_t[:, H:2 * H] + gh[:, H:2 * H])
        n = jnp.tanh(gi_t[:, 2 * H:] + r * gh[:, 2 * H:])
        h = (1.0 - z) * n + z * h
        ys_ref[:, t, :] = h
    hf_ref[...] = h


def _gru(x, wihT, whhT, bih, bhh):
    return pl.pallas_call(
        _gru_body,
        grid=(NB,),
        in_specs=[pl.BlockSpec((BN, T, H), lambda i: (i, 0, 0)),
                  pl.BlockSpec((H, 3 * H), lambda i: (0, 0)),
                  pl.BlockSpec((H, 3 * H), lambda i: (0, 0)),
                  pl.BlockSpec((1, 3 * H), lambda i: (0, 0)),
                  pl.BlockSpec((1, 3 * H), lambda i: (0, 0))],
        out_specs=[pl.BlockSpec((BN, T, H), lambda i: (i, 0, 0)),
                   pl.BlockSpec((BN, H), lambda i: (i, 0))],
        out_shape=[jax.ShapeDtypeStruct((N, T, H), jnp.float32),
                   jax.ShapeDtypeStruct((N, H), jnp.float32)],
        compiler_params=pltpu.CompilerParams(
            dimension_semantics=("parallel",)),
    )(x, wihT, whhT, bih, bhh)


def _dec_body(ys_ref, z_ref, wg_ref, wz_ref, db_ref, c1_ref, b1_ref,
              c2_ref, b2_ref, c3_ref, ls_ref, std_ref, mean_ref, out_ref,
              pad1, pad2):
    zW = jnp.dot(z_ref[...], wz_ref[...], preferred_element_type=jnp.float32)
    h = jnp.dot(ys_ref[...].reshape(BN * T, H), wg_ref[...],
                preferred_element_type=jnp.float32).reshape(BN, T, H)
    h = h + zW[:, None, :] + db_ref[...][None, :, :]

    pad1[...] = jnp.zeros_like(pad1)
    pad1[:, 2:T + 2, :] = h
    acc1 = jnp.zeros((BN * T, C), jnp.float32)
    for k in range(K):
        xk = pad1[:, k:k + T, :].reshape(BN * T, H)
        acc1 += jnp.dot(xk, c1_ref[k], preferred_element_type=jnp.float32)
    h1 = jnp.maximum(acc1 + b1_ref[...], 0.0).reshape(BN, T, C)

    pad2[...] = jnp.zeros_like(pad2)
    pad2[:, 2:T + 2, :] = h1
    acc2 = jnp.zeros((BN * T, C), jnp.float32)
    for k in range(K):
        xk = pad2[:, k:k + T, :].reshape(BN * T, C)
        acc2 += jnp.dot(xk, c2_ref[k], preferred_element_type=jnp.float32)
    h2 = jnp.maximum(acc2 + b2_ref[...], 0.0).reshape(BN, T, C)

    pad1[...] = jnp.zeros_like(pad1)
    pad1[:, 2:T + 2, :] = h2
    acc3 = jnp.zeros((BN, T), jnp.float32)
    for k in range(K):
        acc3 += jnp.sum(pad1[:, k:k + T, :] * c3_ref[0, k][None, None, :],
                        axis=-1)
    acc3 = acc3 + ls_ref[1]
    scale = std_ref[...] * jnp.exp(ls_ref[0])
    out_ref[...] = acc3 * scale + mean_ref[...]


def _decoder(ys, z, wgT, wzT, db, c1T, b1, c2T, b2, c3T, ls2, std, mean):
    return pl.pallas_call(
        _dec_body,
        grid=(NB,),
        in_specs=[pl.BlockSpec((BN, T, H), lambda i: (i, 0, 0)),
                  pl.BlockSpec((BN, H), lambda i: (i, 0)),
                  pl.BlockSpec((H, H), lambda i: (0, 0)),
                  pl.BlockSpec((H, H), lambda i: (0, 0)),
                  pl.BlockSpec((1, H), lambda i: (0, 0)),
                  pl.BlockSpec((K, H, C), lambda i: (0, 0, 0)),
                  pl.BlockSpec((1, C), lambda i: (0, 0)),
                  pl.BlockSpec((K, C, C), lambda i: (0, 0, 0)),
                  pl.BlockSpec((1, C), lambda i: (0, 0)),
                  pl.BlockSpec((1, K, C), lambda i: (0, 0, 0)),
                  pl.BlockSpec(memory_space=pltpu.SMEM),
                  pl.BlockSpec((BN, 1), lambda i: (i, 0)),
                  pl.BlockSpec((BN, 1), lambda i: (i, 0))],
        out_specs=pl.BlockSpec((BN, T), lambda i: (i, 0)),
        out_shape=jax.ShapeDtypeStruct((N, T), jnp.float32),
        scratch_shapes=[pltpu.VMEM((BN, T + 4, C), jnp.float32)] * 2,
        compiler_params=pltpu.CompilerParams(
            dimension_semantics=("arbitrary",)),
    )(ys, z, wgT, wzT, db, c1T, b1, c2T, b2, c3T, ls2, std, mean)


def _head_body(ysf_ref, yse_ref, zf_ref, ze_ref, wbar_ref,
               gw1_ref, gb1_ref, gw2_ref, gb2_ref,
               pf_ref, pe_ref, gs_ref, zsum):
    i = pl.program_id(0)

    @pl.when(i == 0)
    def _init():
        zsum[...] = jnp.zeros_like(zsum)

    pf = jnp.zeros((BN, H), jnp.float32)
    pe = jnp.zeros((BN, H), jnp.float32)
    for t in range(T):
        pf += ysf_ref[:, t, :] * wbar_ref[0, t]
        pe += yse_ref[:, t, :] * wbar_ref[1, t]
    pf_ref[...] = pf
    pe_ref[...] = pe
    zsum[...] += (jnp.sum(zf_ref[...], axis=0, keepdims=True)
                  + jnp.sum(ze_ref[...], axis=0, keepdims=True))

    @pl.when(i == NB - 1)
    def _fin():
        g = zsum[...] * (0.5 / N)
        t1 = jnp.maximum(
            jnp.dot(g, gw1_ref[...], preferred_element_type=jnp.float32)
            + gb1_ref[...], 0.0)
        gs_ref[...] = jnp.dot(t1, gw2_ref[...],
                              preferred_element_type=jnp.float32) + gb2_ref[...]


def _head(ysf, yse, zf, ze, wbar, gw1T, gb1, gw2T, gb2):
    return pl.pallas_call(
        _head_body,
        grid=(NB,),
        in_specs=[pl.BlockSpec((BN, T, H), lambda i: (i, 0, 0)),
                  pl.BlockSpec((BN, T, H), lambda i: (i, 0, 0)),
                  pl.BlockSpec((BN, H), lambda i: (i, 0)),
                  pl.BlockSpec((BN, H), lambda i: (i, 0)),
                  pl.BlockSpec(memory_space=pltpu.SMEM),
                  pl.BlockSpec((H, H), lambda i: (0, 0)),
                  pl.BlockSpec((1, H), lambda i: (0, 0)),
                  pl.BlockSpec((H, H), lambda i: (0, 0)),
                  pl.BlockSpec((1, H), lambda i: (0, 0))],
        out_specs=[pl.BlockSpec((BN, H), lambda i: (i, 0)),
                   pl.BlockSpec((BN, H), lambda i: (i, 0)),
                   pl.BlockSpec((1, H), lambda i: (0, 0))],
        out_shape=[jax.ShapeDtypeStruct((N, H), jnp.float32),
                   jax.ShapeDtypeStruct((N, H), jnp.float32),
                   jax.ShapeDtypeStruct((1, H), jnp.float32)],
        scratch_shapes=[pltpu.VMEM((1, H), jnp.float32)],
        compiler_params=pltpu.CompilerParams(
            dimension_semantics=("arbitrary",)),
    )(ysf, yse, zf, ze, wbar, gw1T, gb1, gw2T, gb2)


SLAB_ROWS = 32 * (N // 16)
TRASH = SLAB_ROWS
ECHUNK = 4096
NCHUNK = E // ECHUNK


SLAB_ELEMS = 32 * N
TRASH_BASE = SLAB_ELEMS


def _adj_body(sff, dff, see, dee, sfe, dfe, adj, s_buf, d_buf, slab):
    c = lax.axis_index("c")
    s = lax.axis_index("s")
    w = c * 16 + s
    lanes = lax.iota(jnp.int32, 16)
    zvec = jnp.zeros((16,), jnp.float32)

    for rel, (src_h, dst_h) in enumerate(((sff, dff), (see, dee), (sfe, dfe))):
        for sub in range(2):
            slab_id = w * 2 + sub
            r0 = slab_id * 32

            def zbody(i, _):
                for k in range(8):
                    slab[pl.ds((i * 8 + k) * 16, 16)] = zvec
                return 0
            lax.fori_loop(0, (SLAB_ELEMS + 16) // 128, zbody, 0)

            def chunk_body(cidx, _):
                pltpu.sync_copy(src_h.at[pl.ds(cidx * ECHUNK, ECHUNK)], s_buf)
                pltpu.sync_copy(dst_h.at[pl.ds(cidx * ECHUNK, ECHUNK)], d_buf)

                def vec_body(v, _):
                    for u in range(4):
                        off = (v * 4 + u) * 16
                        sv = s_buf[pl.ds(off, 16)]
                        dv = d_buf[pl.ds(off, 16)]
                        loc = dv - r0
                        valid = (loc >= 0) & (loc < 32)
                        flat = jnp.where(valid, loc * N + sv,
                                         TRASH_BASE + lanes)
                        cnt, last = plsc.scan_count(flat)
                        plsc.addupdate_scatter(
                            slab, [flat], cnt.astype(jnp.float32), mask=last)
                    return 0
                lax.fori_loop(0, ECHUNK // 64, vec_body, 0)
                return 0
            lax.fori_loop(0, NCHUNK, chunk_body, 0)

            hbm_off = rel * N * N + slab_id * SLAB_ELEMS
            pltpu.sync_copy(slab.at[pl.ds(0, SLAB_ELEMS)],
                            adj.at[pl.ds(hbm_off, SLAB_ELEMS)])


def _build_adj_sc(ei_ff, ei_ee, ei_fe):
    mesh = plsc.VectorSubcoreMesh(core_axis_name="c", subcore_axis_name="s")
    f = pl.kernel(
        _adj_body,
        mesh=mesh,
        out_type=jax.ShapeDtypeStruct((3 * N * N,), jnp.float32),
        scratch_types=[
            pltpu.VMEM((ECHUNK,), jnp.int32),
            pltpu.VMEM((ECHUNK,), jnp.int32),
            pltpu.VMEM((SLAB_ELEMS + 16,), jnp.float32),
        ],
        compiler_params=pltpu.CompilerParams(needs_layout_passes=False,
                                             use_tc_tiling_on_sc=False),
    )
    i32 = jnp.int32
    adj = f(ei_ff[0].astype(i32), ei_ff[1].astype(i32),
            ei_ee[0].astype(i32), ei_ee[1].astype(i32),
            ei_fe[0].astype(i32), ei_fe[1].astype(i32))
    return adj.reshape(3, N, N)


def kernel(encoded_fmri, encoded_eeg, mean_fmri, std_fmri, mean_eeg, std_eeg,
           sage_Wr, sage_Wn, sage_b, gru_Wih, gru_Whh, gru_bih, gru_bhh,
           tproj_W, dec_in_W, dec_in_b, c1W, c1b, c2W, c2b, c3W, c3b,
           log_scale, gW1, gb1, gW2, gb2, ei_ff, ei_ee, ei_fe):
    adj3 = _build_adj_sc(ei_ff, ei_ee, ei_fe).astype(jnp.bfloat16)
    a_ff, a_ee, a_fe = adj3[0], adj3[1], adj3[2]

    xf, xe = encoded_fmri, encoded_eeg
    for l in range(L):
        xf, xe = _sage_layer(
            a_ff, a_ee, a_fe,
            xf.astype(jnp.bfloat16), xe.astype(jnp.bfloat16), xf, xe,
            sage_Wr[l, 0].T, sage_Wn[l, 0].T, sage_b[l, 0].reshape(1, H),
            (sage_Wr[l, 1] + sage_Wr[l, 2]).T, sage_Wn[l, 1].T,
            sage_Wn[l, 2].T, (sage_b[l, 1] + sage_b[l, 2]).reshape(1, H))

    ysf, zf = _gru(xf, gru_Wih[0].T, gru_Whh[0].T,
                   gru_bih[0].reshape(1, 3 * H), gru_bhh[0].reshape(1, 3 * H))
    yse, ze = _gru(xe, gru_Wih[1].T, gru_Whh[1].T,
                   gru_bih[1].reshape(1, 3 * H), gru_bhh[1].reshape(1, 3 * H))

    rf = _decoder(ysf, zf, dec_in_W[0, :, :H].T, dec_in_W[0, :, H:].T,
                  dec_in_b[0].reshape(1, H), c1W[0].transpose(2, 1, 0),
                  c1b[0].reshape(1, C), c2W[0].transpose(2, 1, 0),
                  c2b[0].reshape(1, C), c3W[0].transpose(0, 2, 1),
                  jnp.stack([log_scale[0], c3b[0, 0]]),
                  std_fmri.reshape(N, 1), mean_fmri.reshape(N, 1))
    re = _decoder(yse, ze, dec_in_W[1, :, :H].T, dec_in_W[1, :, H:].T,
                  dec_in_b[1].reshape(1, H), c1W[1].transpose(2, 1, 0),
                  c1b[1].reshape(1, C), c2W[1].transpose(2, 1, 0),
                  c2b[1].reshape(1, C), c3W[1].transpose(0, 2, 1),
                  jnp.stack([log_scale[1], c3b[1, 0]]),
                  std_eeg.reshape(N, 1), mean_eeg.reshape(N, 1))

    wbar = tproj_W.mean(axis=1)
    pf, pe, gs = _head(ysf, yse, zf, ze, wbar, gW1.T, gb1.reshape(1, H),
                       gW2.T, gb2.reshape(1, H))

    return jnp.concatenate([rf.ravel(), re.ravel(), zf.ravel(), ze.ravel(),
                            pf.ravel(), pe.ravel(), gs.ravel()])


# speedup vs baseline: 1.1038x; 1.1038x over previous
import functools

import jax
import jax.numpy as jnp
from jax import lax
from jax.experimental import pallas as pl
from jax.experimental.pallas import tpu as pltpu
from jax.experimental.pallas import tpu_sc as plsc

N, T, H, E, C, K, TP, L = 2048, 16, 128, 32768, 128, 5, 64, 2

BI = 256
BK = 512
NI = N // BI
NK = N // BK
BN = 256
NB = N // BN


def _sage_body(aff_ref, aee_ref, afe_ref, xfs_ref, xes_ref, xfd_ref, xed_ref,
               wrf_ref, wnf_ref, bf_ref, wre_ref, wnee_ref, wnfe_ref, be_ref,
               of_ref, oe_ref,
               accf, acce, accfe, cntf, cnte, cntfe):
    k = pl.program_id(1)

    @pl.when(k == 0)
    def _init():
        accf[...] = jnp.zeros_like(accf)
        acce[...] = jnp.zeros_like(acce)
        accfe[...] = jnp.zeros_like(accfe)
        cntf[...] = jnp.zeros_like(cntf)
        cnte[...] = jnp.zeros_like(cnte)
        cntfe[...] = jnp.zeros_like(cntfe)

    aff = aff_ref[...]
    aee = aee_ref[...]
    afe = afe_ref[...]
    cntf[...] += jnp.sum(aff.astype(jnp.float32), axis=1, keepdims=True)
    cnte[...] += jnp.sum(aee.astype(jnp.float32), axis=1, keepdims=True)
    cntfe[...] += jnp.sum(afe.astype(jnp.float32), axis=1, keepdims=True)
    for t in range(T):
        xf_t = xfs_ref[:, t, :]
        xe_t = xes_ref[:, t, :]
        accf[:, t, :] += jnp.dot(aff, xf_t, preferred_element_type=jnp.float32)
        acce[:, t, :] += jnp.dot(aee, xe_t, preferred_element_type=jnp.float32)
        accfe[:, t, :] += jnp.dot(afe, xf_t, preferred_element_type=jnp.float32)

    @pl.when(k == NK - 1)
    def _fin():
        invf = 1.0 / jnp.maximum(cntf[...], 1.0)
        inve = 1.0 / jnp.maximum(cnte[...], 1.0)
        invfe = 1.0 / jnp.maximum(cntfe[...], 1.0)
        aggf = (accf[...] * invf[:, :, None]).reshape(BI * T, H)
        agge = (acce[...] * inve[:, :, None]).reshape(BI * T, H)
        aggfe = (accfe[...] * invfe[:, :, None]).reshape(BI * T, H)
        xdf = xfd_ref[...].reshape(BI * T, H)
        xde = xed_ref[...].reshape(BI * T, H)
        of = (jnp.dot(xdf, wrf_ref[...], preferred_element_type=jnp.float32)
              + jnp.dot(aggf, wnf_ref[...], preferred_element_type=jnp.float32)
              + bf_ref[...])
        oe = (jnp.dot(xde, wre_ref[...], preferred_element_type=jnp.float32)
              + jnp.dot(agge, wnee_ref[...], preferred_element_type=jnp.float32)
              + jnp.dot(aggfe, wnfe_ref[...], preferred_element_type=jnp.float32)
              + be_ref[...])
        of_ref[...] = jnp.maximum(of, 0.0).reshape(BI, T, H)
        oe_ref[...] = jnp.maximum(0.5 * oe, 0.0).reshape(BI, T, H)


def _sage_layer(a_ff, a_ee, a_fe, xfb, xeb, xf, xe,
                wrf, wnf, bf, wre, wnee, wnfe, be):
    grid = (NI, NK)
    adj_spec = pl.BlockSpec((BI, BK), lambda i, k: (i, k))
    src_spec = pl.BlockSpec((BK, T, H), lambda i, k: (k, 0, 0))
    dst_spec = pl.BlockSpec((BI, T, H), lambda i, k: (i, 0, 0))
    w_spec = pl.BlockSpec((H, H), lambda i, k: (0, 0))
    b_spec = pl.BlockSpec((1, H), lambda i, k: (0, 0))
    out_spec = pl.BlockSpec((BI, T, H), lambda i, k: (i, 0, 0))
    return pl.pallas_call(
        _sage_body,
        grid=grid,
        in_specs=[adj_spec, adj_spec, adj_spec, src_spec, src_spec,
                  dst_spec, dst_spec,
                  w_spec, w_spec, b_spec, w_spec, w_spec, w_spec, b_spec],
        out_specs=[out_spec, out_spec],
        out_shape=[jax.ShapeDtypeStruct((N, T, H), jnp.float32)] * 2,
        scratch_shapes=[pltpu.VMEM((BI, T, H), jnp.float32)] * 3
        + [pltpu.VMEM((BI, 1), jnp.float32)] * 3,
        compiler_params=pltpu.CompilerParams(
            dimension_semantics=("parallel", "arbitrary")),
    )(a_ff, a_ee, a_fe, xfb, xeb, xf, xe, wrf, wnf, bf, wre, wnee, wnfe, be)


def _gru_body(x_ref, wih_ref, whh_ref, bih_ref, bhh_ref, ys_ref, hf_ref):
    x = x_ref[...].reshape(BN * T, H)
    gi = (jnp.dot(x, wih_ref[...], preferred_element_type=jnp.float32)
          + bih_ref[...]).reshape(BN, T, 3 * H)
    h = jnp.zeros((BN, H), jnp.float32)
    for t in range(T):
        gi_t = gi[:, t, :]
        gh = jnp.dot(h, whh_ref[...], preferred_element_type=jnp.float32) \
            + bhh_ref[...]
        r = jax.nn.sigmoid(gi_t[:, :H] + gh[:, :H])
        z = jax.nn.sigmoid(gi_t[:, H:2 * H] + gh[:, H:2 * H])
        n = jnp.tanh(gi_t[:, 2 * H:] + r * gh[:, 2 * H:])
        h = (1.0 - z) * n + z * h
        ys_ref[:, t, :] = h
    hf_ref[...] = h


def _gru(x, wihT, whhT, bih, bhh):
    return pl.pallas_call(
        _gru_body,
        grid=(NB,),
        in_specs=[pl.BlockSpec((BN, T, H), lambda i: (i, 0, 0)),
                  pl.BlockSpec((H, 3 * H), lambda i: (0, 0)),
                  pl.BlockSpec((H, 3 * H), lambda i: (0, 0)),
                  pl.BlockSpec((1, 3 * H), lambda i: (0, 0)),
                  pl.BlockSpec((1, 3 * H), lambda i: (0, 0))],
        out_specs=[pl.BlockSpec((BN, T, H), lambda i: (i, 0, 0)),
                   pl.BlockSpec((BN, H), lambda i: (i, 0))],
        out_shape=[jax.ShapeDtypeStruct((N, T, H), jnp.float32),
                   jax.ShapeDtypeStruct((N, H), jnp.float32)],
        compiler_params=pltpu.CompilerParams(
            dimension_semantics=("parallel",)),
    )(x, wihT, whhT, bih, bhh)


def _dec_body(ys_ref, z_ref, wg_ref, wz_ref, db_ref, c1_ref, b1_ref,
              c2_ref, b2_ref, c3_ref, ls_ref, std_ref, mean_ref, out_ref,
              pad1, pad2):
    zW = jnp.dot(z_ref[...], wz_ref[...], preferred_element_type=jnp.float32)
    h = jnp.dot(ys_ref[...].reshape(BN * T, H), wg_ref[...],
                preferred_element_type=jnp.float32).reshape(BN, T, H)
    h = h + zW[:, None, :] + db_ref[...][None, :, :]

    pad1[...] = jnp.zeros_like(pad1)
    pad1[:, 2:T + 2, :] = h
    acc1 = jnp.zeros((BN * T, C), jnp.float32)
    for k in range(K):
        xk = pad1[:, k:k + T, :].reshape(BN * T, H)
        acc1 += jnp.dot(xk, c1_ref[k], preferred_element_type=jnp.float32)
    h1 = jnp.maximum(acc1 + b1_ref[...], 0.0).reshape(BN, T, C)

    pad2[...] = jnp.zeros_like(pad2)
    pad2[:, 2:T + 2, :] = h1
    acc2 = jnp.zeros((BN * T, C), jnp.float32)
    for k in range(K):
        xk = pad2[:, k:k + T, :].reshape(BN * T, C)
        acc2 += jnp.dot(xk, c2_ref[k], preferred_element_type=jnp.float32)
    h2 = jnp.maximum(acc2 + b2_ref[...], 0.0).reshape(BN, T, C)

    pad1[...] = jnp.zeros_like(pad1)
    pad1[:, 2:T + 2, :] = h2
    acc3 = jnp.zeros((BN, T), jnp.float32)
    for k in range(K):
        acc3 += jnp.sum(pad1[:, k:k + T, :] * c3_ref[0, k][None, None, :],
                        axis=-1)
    acc3 = acc3 + ls_ref[1]
    scale = std_ref[...] * jnp.exp(ls_ref[0])
    out_ref[...] = acc3 * scale + mean_ref[...]


def _decoder(ys, z, wgT, wzT, db, c1T, b1, c2T, b2, c3T, ls2, std, mean):
    return pl.pallas_call(
        _dec_body,
        grid=(NB,),
        in_specs=[pl.BlockSpec((BN, T, H), lambda i: (i, 0, 0)),
                  pl.BlockSpec((BN, H), lambda i: (i, 0)),
                  pl.BlockSpec((H, H), lambda i: (0, 0)),
                  pl.BlockSpec((H, H), lambda i: (0, 0)),
                  pl.BlockSpec((1, H), lambda i: (0, 0)),
                  pl.BlockSpec((K, H, C), lambda i: (0, 0, 0)),
                  pl.BlockSpec((1, C), lambda i: (0, 0)),
                  pl.BlockSpec((K, C, C), lambda i: (0, 0, 0)),
                  pl.BlockSpec((1, C), lambda i: (0, 0)),
                  pl.BlockSpec((1, K, C), lambda i: (0, 0, 0)),
                  pl.BlockSpec(memory_space=pltpu.SMEM),
                  pl.BlockSpec((BN, 1), lambda i: (i, 0)),
                  pl.BlockSpec((BN, 1), lambda i: (i, 0))],
        out_specs=pl.BlockSpec((BN, T), lambda i: (i, 0)),
        out_shape=jax.ShapeDtypeStruct((N, T), jnp.float32),
        scratch_shapes=[pltpu.VMEM((BN, T + 4, C), jnp.float32)] * 2,
        compiler_params=pltpu.CompilerParams(
            dimension_semantics=("arbitrary",)),
    )(ys, z, wgT, wzT, db, c1T, b1, c2T, b2, c3T, ls2, std, mean)


def _head_body(ysf_ref, yse_ref, zf_ref, ze_ref, wbar_ref,
               gw1_ref, gb1_ref, gw2_ref, gb2_ref,
               pf_ref, pe_ref, gs_ref, zsum):
    i = pl.program_id(0)

    @pl.when(i == 0)
    def _init():
        zsum[...] = jnp.zeros_like(zsum)

    pf = jnp.zeros((BN, H), jnp.float32)
    pe = jnp.zeros((BN, H), jnp.float32)
    for t in range(T):
        pf += ysf_ref[:, t, :] * wbar_ref[0, t]
        pe += yse_ref[:, t, :] * wbar_ref[1, t]
    pf_ref[...] = pf
    pe_ref[...] = pe
    zsum[...] += (jnp.sum(zf_ref[...], axis=0, keepdims=True)
                  + jnp.sum(ze_ref[...], axis=0, keepdims=True))

    @pl.when(i == NB - 1)
    def _fin():
        g = zsum[...] * (0.5 / N)
        t1 = jnp.maximum(
            jnp.dot(g, gw1_ref[...], preferred_element_type=jnp.float32)
            + gb1_ref[...], 0.0)
        gs_ref[...] = jnp.dot(t1, gw2_ref[...],
                              preferred_element_type=jnp.float32) + gb2_ref[...]


def _head(ysf, yse, zf, ze, wbar, gw1T, gb1, gw2T, gb2):
    return pl.pallas_call(
        _head_body,
        grid=(NB,),
        in_specs=[pl.BlockSpec((BN, T, H), lambda i: (i, 0, 0)),
                  pl.BlockSpec((BN, T, H), lambda i: (i, 0, 0)),
                  pl.BlockSpec((BN, H), lambda i: (i, 0)),
                  pl.BlockSpec((BN, H), lambda i: (i, 0)),
                  pl.BlockSpec(memory_space=pltpu.SMEM),
                  pl.BlockSpec((H, H), lambda i: (0, 0)),
                  pl.BlockSpec((1, H), lambda i: (0, 0)),
                  pl.BlockSpec((H, H), lambda i: (0, 0)),
                  pl.BlockSpec((1, H), lambda i: (0, 0))],
        out_specs=[pl.BlockSpec((BN, H), lambda i: (i, 0)),
                   pl.BlockSpec((BN, H), lambda i: (i, 0)),
                   pl.BlockSpec((1, H), lambda i: (0, 0))],
        out_shape=[jax.ShapeDtypeStruct((N, H), jnp.float32),
                   jax.ShapeDtypeStruct((N, H), jnp.float32),
                   jax.ShapeDtypeStruct((1, H), jnp.float32)],
        scratch_shapes=[pltpu.VMEM((1, H), jnp.float32)],
        compiler_params=pltpu.CompilerParams(
            dimension_semantics=("arbitrary",)),
    )(ysf, yse, zf, ze, wbar, gw1T, gb1, gw2T, gb2)


SLAB_ROWS = 32 * (N // 16)
TRASH = SLAB_ROWS
ECHUNK = 4096
NCHUNK = E // ECHUNK


SLAB_ELEMS = 32 * N
TRASH_BASE = SLAB_ELEMS


def _adj_body(sff, dff, see, dee, sfe, dfe, adj, s_buf, d_buf, slab):
    c = lax.axis_index("c")
    s = lax.axis_index("s")
    w = c * 16 + s
    lanes = lax.iota(jnp.int32, 16)
    zvec = jnp.zeros((16,), jnp.float32)

    for rel, (src_h, dst_h) in enumerate(((sff, dff), (see, dee), (sfe, dfe))):
        for sub in range(2):
            slab_id = w * 2 + sub
            r0 = slab_id * 32

            def zbody(i, _):
                for k in range(8):
                    slab[pl.ds((i * 8 + k) * 16, 16)] = zvec
                return 0
            lax.fori_loop(0, (SLAB_ELEMS + 16) // 128, zbody, 0)

            def chunk_body(cidx, _):
                pltpu.sync_copy(src_h.at[pl.ds(cidx * ECHUNK, ECHUNK)], s_buf)
                pltpu.sync_copy(dst_h.at[pl.ds(cidx * ECHUNK, ECHUNK)], d_buf)

                def vec_body(v, _):
                    for u in range(4):
                        off = (v * 4 + u) * 16
                        sv = s_buf[pl.ds(off, 16)]
                        dv = d_buf[pl.ds(off, 16)]
                        loc = dv - r0
                        valid = (loc >= 0) & (loc < 32)
                        flat = jnp.where(valid, loc * N + sv,
                                         TRASH_BASE + lanes)
                        cnt, last = plsc.scan_count(flat)
                        plsc.addupdate_scatter(
                            slab, [flat], cnt.astype(jnp.float32), mask=last)
                    return 0
                lax.fori_loop(0, ECHUNK // 64, vec_body, 0)
                return 0
            lax.fori_loop(0, NCHUNK, chunk_body, 0)

            hbm_off = rel * N * N + slab_id * SLAB_ELEMS
            pltpu.sync_copy(slab.at[pl.ds(0, SLAB_ELEMS)],
                            adj.at[pl.ds(hbm_off, SLAB_ELEMS)])


def _build_adj_sc(ei_ff, ei_ee, ei_fe):
    mesh = plsc.VectorSubcoreMesh(core_axis_name="c", subcore_axis_name="s")
    f = pl.kernel(
        _adj_body,
        mesh=mesh,
        out_type=jax.ShapeDtypeStruct((3 * N * N,), jnp.float32),
        scratch_types=[
            pltpu.VMEM((ECHUNK,), jnp.int32),
            pltpu.VMEM((ECHUNK,), jnp.int32),
            pltpu.VMEM((SLAB_ELEMS + 16,), jnp.float32),
        ],
        compiler_params=pltpu.CompilerParams(needs_layout_passes=False,
                                             use_tc_tiling_on_sc=False),
    )
    i32 = jnp.int32
    adj = f(ei_ff[0].astype(i32), ei_ff[1].astype(i32),
            ei_ee[0].astype(i32), ei_ee[1].astype(i32),
            ei_fe[0].astype(i32), ei_fe[1].astype(i32))
    return adj.reshape(3, N, N)


def kernel(encoded_fmri, encoded_eeg, mean_fmri, std_fmri, mean_eeg, std_eeg,
           sage_Wr, sage_Wn, sage_b, gru_Wih, gru_Whh, gru_bih, gru_bhh,
           tproj_W, dec_in_W, dec_in_b, c1W, c1b, c2W, c2b, c3W, c3b,
           log_scale, gW1, gb1, gW2, gb2, ei_ff, ei_ee, ei_fe):
    adj3 = _build_adj_sc(ei_ff, ei_ee, ei_fe)
    a_ff, a_ee, a_fe = adj3[0], adj3[1], adj3[2]

    xf, xe = encoded_fmri, encoded_eeg
    for l in range(L):
        xf, xe = _sage_layer(
            a_ff, a_ee, a_fe, xf, xe, xf, xe,
            sage_Wr[l, 0].T, sage_Wn[l, 0].T, sage_b[l, 0].reshape(1, H),
            (sage_Wr[l, 1] + sage_Wr[l, 2]).T, sage_Wn[l, 1].T,
            sage_Wn[l, 2].T, (sage_b[l, 1] + sage_b[l, 2]).reshape(1, H))

    ysf, zf = _gru(xf, gru_Wih[0].T, gru_Whh[0].T,
                   gru_bih[0].reshape(1, 3 * H), gru_bhh[0].reshape(1, 3 * H))
    yse, ze = _gru(xe, gru_Wih[1].T, gru_Whh[1].T,
                   gru_bih[1].reshape(1, 3 * H), gru_bhh[1].reshape(1, 3 * H))

    rf = _decoder(ysf, zf, dec_in_W[0, :, :H].T, dec_in_W[0, :, H:].T,
                  dec_in_b[0].reshape(1, H), c1W[0].transpose(2, 1, 0),
                  c1b[0].reshape(1, C), c2W[0].transpose(2, 1, 0),
                  c2b[0].reshape(1, C), c3W[0].transpose(0, 2, 1),
                  jnp.stack([log_scale[0], c3b[0, 0]]),
                  std_fmri.reshape(N, 1), mean_fmri.reshape(N, 1))
    re = _decoder(yse, ze, dec_in_W[1, :, :H].T, dec_in_W[1, :, H:].T,
                  dec_in_b[1].reshape(1, H), c1W[1].transpose(2, 1, 0),
                  c1b[1].reshape(1, C), c2W[1].transpose(2, 1, 0),
                  c2b[1].reshape(1, C), c3W[1].transpose(0, 2, 1),
                  jnp.stack([log_scale[1], c3b[1, 0]]),
                  std_eeg.reshape(N, 1), mean_eeg.reshape(N, 1))

    wbar = tproj_W.mean(axis=1)
    pf, pe, gs = _head(ysf, yse, zf, ze, wbar, gW1.T, gb1.reshape(1, H),
                       gW2.T, gb2.reshape(1, H))

    return jnp.concatenate([rf.ravel(), re.ravel(), zf.ravel(), ze.ravel(),
                            pf.ravel(), pe.ravel(), gs.ravel()])


# speedup vs baseline: 1.3769x; 1.2474x over previous
import functools

import jax
import jax.numpy as jnp
from jax import lax
from jax.experimental import pallas as pl
from jax.experimental.pallas import tpu as pltpu
from jax.experimental.pallas import tpu_sc as plsc

N, T, H, E, C, K, TP, L = 2048, 16, 128, 32768, 128, 5, 64, 2

BI = 256
BK = 512
NI = N // BI
NK = N // BK
BN = 256
NB = N // BN


def _agg_body(aff_ref, aee_ref, afe_ref, xfs_ref, xes_ref,
              cf_ref, ce_ref, cfe_ref, of_ref, oe_ref, ofe_ref,
              accf, acce, accfe):
    k = pl.program_id(1)

    @pl.when(k == 0)
    def _init():
        accf[...] = jnp.zeros_like(accf)
        acce[...] = jnp.zeros_like(acce)
        accfe[...] = jnp.zeros_like(accfe)

    xf = xfs_ref[...]
    xe = xes_ref[...]
    accf[...] += jnp.dot(aff_ref[...], xf, preferred_element_type=jnp.float32)
    acce[...] += jnp.dot(aee_ref[...], xe, preferred_element_type=jnp.float32)
    accfe[...] += jnp.dot(afe_ref[...], xf, preferred_element_type=jnp.float32)

    @pl.when(k == NK - 1)
    def _fin():
        of_ref[...] = accf[...] * (1.0 / jnp.maximum(cf_ref[...], 1.0))
        oe_ref[...] = acce[...] * (1.0 / jnp.maximum(ce_ref[...], 1.0))
        ofe_ref[...] = accfe[...] * (1.0 / jnp.maximum(cfe_ref[...], 1.0))


def _agg(a_ff, a_ee, a_fe, xf_flat, xe_flat, cf, ce, cfe):
    adj_spec = pl.BlockSpec((BI, BK), lambda i, k: (i, k))
    src_spec = pl.BlockSpec((BK, T * H), lambda i, k: (k, 0))
    cnt_spec = pl.BlockSpec((BI, 1), lambda i, k: (i, 0))
    out_spec = pl.BlockSpec((BI, T * H), lambda i, k: (i, 0))
    return pl.pallas_call(
        _agg_body,
        grid=(NI, NK),
        in_specs=[adj_spec, adj_spec, adj_spec, src_spec, src_spec,
                  cnt_spec, cnt_spec, cnt_spec],
        out_specs=[out_spec] * 3,
        out_shape=[jax.ShapeDtypeStruct((N, T * H), jnp.float32)] * 3,
        scratch_shapes=[pltpu.VMEM((BI, T * H), jnp.float32)] * 3,
        compiler_params=pltpu.CompilerParams(
            dimension_semantics=("parallel", "arbitrary")),
    )(a_ff, a_ee, a_fe, xf_flat, xe_flat, cf, ce, cfe)


def _lin_body(xf_ref, xe_ref, agf_ref, age_ref, agfe_ref,
              wrf_ref, wnf_ref, bf_ref, wre_ref, wnee_ref, wnfe_ref, be_ref,
              of_ref, oe_ref):
    xdf = xf_ref[...].reshape(BI * T, H)
    xde = xe_ref[...].reshape(BI * T, H)
    aggf = agf_ref[...].reshape(BI * T, H)
    agge = age_ref[...].reshape(BI * T, H)
    aggfe = agfe_ref[...].reshape(BI * T, H)
    of = (jnp.dot(xdf, wrf_ref[...], preferred_element_type=jnp.float32)
          + jnp.dot(aggf, wnf_ref[...], preferred_element_type=jnp.float32)
          + bf_ref[...])
    oe = (jnp.dot(xde, wre_ref[...], preferred_element_type=jnp.float32)
          + jnp.dot(agge, wnee_ref[...], preferred_element_type=jnp.float32)
          + jnp.dot(aggfe, wnfe_ref[...], preferred_element_type=jnp.float32)
          + be_ref[...])
    of_ref[...] = jnp.maximum(of, 0.0).reshape(BI, T, H)
    oe_ref[...] = jnp.maximum(0.5 * oe, 0.0).reshape(BI, T, H)


def _lin(xf, xe, agf, age, agfe, wrf, wnf, bf, wre, wnee, wnfe, be):
    x_spec = pl.BlockSpec((BI, T, H), lambda i: (i, 0, 0))
    w_spec = pl.BlockSpec((H, H), lambda i: (0, 0))
    b_spec = pl.BlockSpec((1, H), lambda i: (0, 0))
    return pl.pallas_call(
        _lin_body,
        grid=(NI,),
        in_specs=[x_spec] * 5 + [w_spec, w_spec, b_spec,
                                 w_spec, w_spec, w_spec, b_spec],
        out_specs=[x_spec, x_spec],
        out_shape=[jax.ShapeDtypeStruct((N, T, H), jnp.float32)] * 2,
        compiler_params=pltpu.CompilerParams(
            dimension_semantics=("parallel",)),
    )(xf, xe, agf, age, agfe, wrf, wnf, bf, wre, wnee, wnfe, be)


def _sage_layer(a_ff, a_ee, a_fe, xf, xe, cf, ce, cfe,
                wrf, wnf, bf, wre, wnee, wnfe, be):
    agf, age, agfe = _agg(a_ff, a_ee, a_fe, xf.reshape(N, T * H),
                          xe.reshape(N, T * H), cf, ce, cfe)
    return _lin(xf, xe, agf.reshape(N, T, H), age.reshape(N, T, H),
                agfe.reshape(N, T, H), wrf, wnf, bf, wre, wnee, wnfe, be)


def _gru_body(x_ref, wih_ref, whh_ref, bih_ref, bhh_ref, ys_ref, hf_ref):
    x = x_ref[...].reshape(BN * T, H)
    gi = (jnp.dot(x, wih_ref[...], preferred_element_type=jnp.float32)
          + bih_ref[...]).reshape(BN, T, 3 * H)
    h = jnp.zeros((BN, H), jnp.float32)
    for t in range(T):
        gi_t = gi[:, t, :]
        gh = jnp.dot(h, whh_ref[...], preferred_element_type=jnp.float32) \
            + bhh_ref[...]
        r = jax.nn.sigmoid(gi_t[:, :H] + gh[:, :H])
        z = jax.nn.sigmoid(gi_t[:, H:2 * H] + gh[:, H:2 * H])
        n = jnp.tanh(gi_t[:, 2 * H:] + r * gh[:, 2 * H:])
        h = (1.0 - z) * n + z * h
        ys_ref[t] = h
    hf_ref[...] = h


def _gru(x, wihT, whhT, bih, bhh):
    return pl.pallas_call(
        _gru_body,
        grid=(NB,),
        in_specs=[pl.BlockSpec((BN, T, H), lambda i: (i, 0, 0)),
                  pl.BlockSpec((H, 3 * H), lambda i: (0, 0)),
                  pl.BlockSpec((H, 3 * H), lambda i: (0, 0)),
                  pl.BlockSpec((1, 3 * H), lambda i: (0, 0)),
                  pl.BlockSpec((1, 3 * H), lambda i: (0, 0))],
        out_specs=[pl.BlockSpec((T, BN, H), lambda i: (0, i, 0)),
                   pl.BlockSpec((BN, H), lambda i: (i, 0))],
        out_shape=[jax.ShapeDtypeStruct((T, N, H), jnp.float32),
                   jax.ShapeDtypeStruct((N, H), jnp.float32)],
        compiler_params=pltpu.CompilerParams(
            dimension_semantics=("parallel",)),
    )(x, wihT, whhT, bih, bhh)


def _dec_body(ys_ref, z_ref, wg_ref, wz_ref, db_ref, c1_ref, b1_ref,
              c2_ref, b2_ref, c3_ref, ls_ref, std_ref, mean_ref, out_ref,
              pad1, pad2):
    zW = jnp.dot(z_ref[...], wz_ref[...], preferred_element_type=jnp.float32)
    h = jnp.dot(ys_ref[...].reshape(T * BN, H), wg_ref[...],
                preferred_element_type=jnp.float32).reshape(T, BN, H)
    h = h + zW[None, :, :] + db_ref[...][None, :, :]

    pad1[...] = jnp.zeros_like(pad1)
    pad1[2:T + 2] = h
    acc1 = jnp.zeros((T * BN, C), jnp.float32)
    for k in range(K):
        xk = pad1[k:k + T].reshape(T * BN, H)
        acc1 += jnp.dot(xk, c1_ref[k], preferred_element_type=jnp.float32)
    h1 = jnp.maximum(acc1 + b1_ref[...], 0.0).reshape(T, BN, C)

    pad2[...] = jnp.zeros_like(pad2)
    pad2[2:T + 2] = h1
    acc2 = jnp.zeros((T * BN, C), jnp.float32)
    for k in range(K):
        xk = pad2[k:k + T].reshape(T * BN, C)
        acc2 += jnp.dot(xk, c2_ref[k], preferred_element_type=jnp.float32)
    h2 = jnp.maximum(acc2 + b2_ref[...], 0.0).reshape(T, BN, C)

    pad1[...] = jnp.zeros_like(pad1)
    pad1[2:T + 2] = h2
    tmp = jnp.zeros((T, BN, C), jnp.float32)
    for k in range(K):
        tmp += pad1[k:k + T] * c3_ref[0, k][None, None, :]
    acc3 = jnp.sum(tmp, axis=-1) + ls_ref[1]
    scale = std_ref[...] * jnp.exp(ls_ref[0])
    out_ref[...] = acc3 * scale + mean_ref[...]


def _decoder(ys, z, wgT, wzT, db, c1T, b1, c2T, b2, c3T, ls2, std, mean):
    return pl.pallas_call(
        _dec_body,
        grid=(NB,),
        in_specs=[pl.BlockSpec((T, BN, H), lambda i: (0, i, 0)),
                  pl.BlockSpec((BN, H), lambda i: (i, 0)),
                  pl.BlockSpec((H, H), lambda i: (0, 0)),
                  pl.BlockSpec((H, H), lambda i: (0, 0)),
                  pl.BlockSpec((1, H), lambda i: (0, 0)),
                  pl.BlockSpec((K, H, C), lambda i: (0, 0, 0)),
                  pl.BlockSpec((1, C), lambda i: (0, 0)),
                  pl.BlockSpec((K, C, C), lambda i: (0, 0, 0)),
                  pl.BlockSpec((1, C), lambda i: (0, 0)),
                  pl.BlockSpec((1, K, C), lambda i: (0, 0, 0)),
                  pl.BlockSpec(memory_space=pltpu.SMEM),
                  pl.BlockSpec((1, BN), lambda i: (0, i)),
                  pl.BlockSpec((1, BN), lambda i: (0, i))],
        out_specs=pl.BlockSpec((T, BN), lambda i: (0, i)),
        out_shape=jax.ShapeDtypeStruct((T, N), jnp.float32),
        scratch_shapes=[pltpu.VMEM((T + 4, BN, C), jnp.float32)] * 2,
        compiler_params=pltpu.CompilerParams(
            dimension_semantics=("arbitrary",)),
    )(ys, z, wgT, wzT, db, c1T, b1, c2T, b2, c3T, ls2, std, mean)


def _head_body(ysf_ref, yse_ref, zf_ref, ze_ref, wbar_ref,
               gw1_ref, gb1_ref, gw2_ref, gb2_ref,
               pf_ref, pe_ref, gs_ref, zsum):
    i = pl.program_id(0)

    @pl.when(i == 0)
    def _init():
        zsum[...] = jnp.zeros_like(zsum)

    pf = jnp.zeros((BN, H), jnp.float32)
    pe = jnp.zeros((BN, H), jnp.float32)
    for t in range(T):
        pf += ysf_ref[t] * wbar_ref[0, t]
        pe += yse_ref[t] * wbar_ref[1, t]
    pf_ref[...] = pf
    pe_ref[...] = pe
    zsum[...] += (jnp.sum(zf_ref[...], axis=0, keepdims=True)
                  + jnp.sum(ze_ref[...], axis=0, keepdims=True))

    @pl.when(i == NB - 1)
    def _fin():
        g = zsum[...] * (0.5 / N)
        t1 = jnp.maximum(
            jnp.dot(g, gw1_ref[...], preferred_element_type=jnp.float32)
            + gb1_ref[...], 0.0)
        gs_ref[...] = jnp.dot(t1, gw2_ref[...],
                              preferred_element_type=jnp.float32) + gb2_ref[...]


def _head(ysf, yse, zf, ze, wbar, gw1T, gb1, gw2T, gb2):
    return pl.pallas_call(
        _head_body,
        grid=(NB,),
        in_specs=[pl.BlockSpec((T, BN, H), lambda i: (0, i, 0)),
                  pl.BlockSpec((T, BN, H), lambda i: (0, i, 0)),
                  pl.BlockSpec((BN, H), lambda i: (i, 0)),
                  pl.BlockSpec((BN, H), lambda i: (i, 0)),
                  pl.BlockSpec(memory_space=pltpu.SMEM),
                  pl.BlockSpec((H, H), lambda i: (0, 0)),
                  pl.BlockSpec((1, H), lambda i: (0, 0)),
                  pl.BlockSpec((H, H), lambda i: (0, 0)),
                  pl.BlockSpec((1, H), lambda i: (0, 0))],
        out_specs=[pl.BlockSpec((BN, H), lambda i: (i, 0)),
                   pl.BlockSpec((BN, H), lambda i: (i, 0)),
                   pl.BlockSpec((1, H), lambda i: (0, 0))],
        out_shape=[jax.ShapeDtypeStruct((N, H), jnp.float32),
                   jax.ShapeDtypeStruct((N, H), jnp.float32),
                   jax.ShapeDtypeStruct((1, H), jnp.float32)],
        scratch_shapes=[pltpu.VMEM((1, H), jnp.float32)],
        compiler_params=pltpu.CompilerParams(
            dimension_semantics=("arbitrary",)),
    )(ysf, yse, zf, ze, wbar, gw1T, gb1, gw2T, gb2)


SLAB_ROWS = 32 * (N // 16)
TRASH = SLAB_ROWS
ECHUNK = 4096
NCHUNK = E // ECHUNK


SLAB_ELEMS = 32 * N
TRASH_BASE = SLAB_ELEMS


def _adj_body(sff, dff, see, dee, sfe, dfe, adj, cnt, s_buf, d_buf, slab,
              cslab):
    c = lax.axis_index("c")
    s = lax.axis_index("s")
    w = c * 16 + s
    lanes = lax.iota(jnp.int32, 16)
    zvec = jnp.zeros((16,), jnp.float32)

    for rel, (src_h, dst_h) in enumerate(((sff, dff), (see, dee), (sfe, dfe))):
        for sub in range(2):
            slab_id = w * 2 + sub
            r0 = slab_id * 32

            def zbody(i, _):
                for k in range(8):
                    slab[pl.ds((i * 8 + k) * 16, 16)] = zvec
                return 0
            lax.fori_loop(0, (SLAB_ELEMS + 16) // 128, zbody, 0)
            for k in range(3):
                cslab[pl.ds(k * 16, 16)] = zvec

            def chunk_body(cidx, _):
                pltpu.sync_copy(src_h.at[pl.ds(cidx * ECHUNK, ECHUNK)], s_buf)
                pltpu.sync_copy(dst_h.at[pl.ds(cidx * ECHUNK, ECHUNK)], d_buf)

                def vec_body(v, _):
                    for u in range(4):
                        off = (v * 4 + u) * 16
                        sv = s_buf[pl.ds(off, 16)]
                        dv = d_buf[pl.ds(off, 16)]
                        loc = dv - r0
                        valid = (loc >= 0) & (loc < 32)
                        flat = jnp.where(valid, loc * N + sv,
                                         TRASH_BASE + lanes)
                        m, last = plsc.scan_count(flat)
                        plsc.addupdate_scatter(
                            slab, [flat], m.astype(jnp.float32), mask=last)
                        flatc = jnp.where(valid, loc, 32 + lanes)
                        mc, lastc = plsc.scan_count(flatc)
                        plsc.addupdate_scatter(
                            cslab, [flatc], mc.astype(jnp.float32),
                            mask=lastc)
                    return 0
                lax.fori_loop(0, ECHUNK // 64, vec_body, 0)
                return 0
            lax.fori_loop(0, NCHUNK, chunk_body, 0)

            hbm_off = rel * N * N + slab_id * SLAB_ELEMS
            pltpu.sync_copy(slab.at[pl.ds(0, SLAB_ELEMS)],
                            adj.at[pl.ds(hbm_off, SLAB_ELEMS)])
            pltpu.sync_copy(cslab.at[pl.ds(0, 32)],
                            cnt.at[pl.ds(rel * N + slab_id * 32, 32)])


def _build_adj_sc(ei_ff, ei_ee, ei_fe):
    mesh = plsc.VectorSubcoreMesh(core_axis_name="c", subcore_axis_name="s")
    f = pl.kernel(
        _adj_body,
        mesh=mesh,
        out_type=[jax.ShapeDtypeStruct((3 * N * N,), jnp.float32),
                  jax.ShapeDtypeStruct((3 * N,), jnp.float32)],
        scratch_types=[
            pltpu.VMEM((ECHUNK,), jnp.int32),
            pltpu.VMEM((ECHUNK,), jnp.int32),
            pltpu.VMEM((SLAB_ELEMS + 16,), jnp.float32),
            pltpu.VMEM((48,), jnp.float32),
        ],
        compiler_params=pltpu.CompilerParams(needs_layout_passes=False,
                                             use_tc_tiling_on_sc=False),
    )
    i32 = jnp.int32
    adj, cnt = f(ei_ff[0].astype(i32), ei_ff[1].astype(i32),
                 ei_ee[0].astype(i32), ei_ee[1].astype(i32),
                 ei_fe[0].astype(i32), ei_fe[1].astype(i32))
    return adj.reshape(3, N, N), cnt.reshape(3, N)


def kernel(encoded_fmri, encoded_eeg, mean_fmri, std_fmri, mean_eeg, std_eeg,
           sage_Wr, sage_Wn, sage_b, gru_Wih, gru_Whh, gru_bih, gru_bhh,
           tproj_W, dec_in_W, dec_in_b, c1W, c1b, c2W, c2b, c3W, c3b,
           log_scale, gW1, gb1, gW2, gb2, ei_ff, ei_ee, ei_fe):
    adj3, cnt3 = _build_adj_sc(ei_ff, ei_ee, ei_fe)
    a_ff, a_ee, a_fe = adj3[0], adj3[1], adj3[2]
    cf, ce, cfe = (cnt3[0].reshape(N, 1), cnt3[1].reshape(N, 1),
                   cnt3[2].reshape(N, 1))

    xf, xe = encoded_fmri, encoded_eeg
    for l in range(L):
        xf, xe = _sage_layer(
            a_ff, a_ee, a_fe, xf, xe, cf, ce, cfe,
            sage_Wr[l, 0].T, sage_Wn[l, 0].T, sage_b[l, 0].reshape(1, H),
            (sage_Wr[l, 1] + sage_Wr[l, 2]).T, sage_Wn[l, 1].T,
            sage_Wn[l, 2].T, (sage_b[l, 1] + sage_b[l, 2]).reshape(1, H))

    ysf, zf = _gru(xf, gru_Wih[0].T, gru_Whh[0].T,
                   gru_bih[0].reshape(1, 3 * H), gru_bhh[0].reshape(1, 3 * H))
    yse, ze = _gru(xe, gru_Wih[1].T, gru_Whh[1].T,
                   gru_bih[1].reshape(1, 3 * H), gru_bhh[1].reshape(1, 3 * H))

    rf = _decoder(ysf, zf, dec_in_W[0, :, :H].T, dec_in_W[0, :, H:].T,
                  dec_in_b[0].reshape(1, H), c1W[0].transpose(2, 1, 0),
                  c1b[0].reshape(1, C), c2W[0].transpose(2, 1, 0),
                  c2b[0].reshape(1, C), c3W[0].transpose(0, 2, 1),
                  jnp.stack([log_scale[0], c3b[0, 0]]),
                  std_fmri.reshape(1, N), mean_fmri.reshape(1, N))
    re = _decoder(yse, ze, dec_in_W[1, :, :H].T, dec_in_W[1, :, H:].T,
                  dec_in_b[1].reshape(1, H), c1W[1].transpose(2, 1, 0),
                  c1b[1].reshape(1, C), c2W[1].transpose(2, 1, 0),
                  c2b[1].reshape(1, C), c3W[1].transpose(0, 2, 1),
                  jnp.stack([log_scale[1], c3b[1, 0]]),
                  std_eeg.reshape(1, N), mean_eeg.reshape(1, N))

    wbar = tproj_W.mean(axis=1)
    pf, pe, gs = _head(ysf, yse, zf, ze, wbar, gW1.T, gb1.reshape(1, H),
                       gW2.T, gb2.reshape(1, H))

    return jnp.concatenate([rf.T.ravel(), re.T.ravel(), zf.ravel(),
                            ze.ravel(), pf.ravel(), pe.ravel(), gs.ravel()])


# speedup vs baseline: 1.3847x; 1.0057x over previous
import functools

import jax
import jax.numpy as jnp
from jax import lax
from jax.experimental import pallas as pl
from jax.experimental.pallas import tpu as pltpu
from jax.experimental.pallas import tpu_sc as plsc

N, T, H, E, C, K, TP, L = 2048, 16, 128, 32768, 128, 5, 64, 2

BI = 256
BK = 512
NI = N // BI
NK = N // BK
BN = 256
NB = N // BN


def _agg_body(aff_ref, aee_ref, afe_ref, xfs_ref, xes_ref,
              of_ref, oe_ref, ofe_ref, accf, acce, accfe, cntf, cnte, cntfe):
    k = pl.program_id(1)

    @pl.when(k == 0)
    def _init():
        accf[...] = jnp.zeros_like(accf)
        acce[...] = jnp.zeros_like(acce)
        accfe[...] = jnp.zeros_like(accfe)
        cntf[...] = jnp.zeros_like(cntf)
        cnte[...] = jnp.zeros_like(cnte)
        cntfe[...] = jnp.zeros_like(cntfe)

    xf = xfs_ref[...]
    xe = xes_ref[...]
    ones = jnp.ones((BK, 128), jnp.float32)
    aff = aff_ref[...]
    aee = aee_ref[...]
    afe = afe_ref[...]
    accf[...] += jnp.dot(aff, xf, preferred_element_type=jnp.float32)
    acce[...] += jnp.dot(aee, xe, preferred_element_type=jnp.float32)
    accfe[...] += jnp.dot(afe, xf, preferred_element_type=jnp.float32)
    cntf[...] += jnp.dot(aff, ones, preferred_element_type=jnp.float32)
    cnte[...] += jnp.dot(aee, ones, preferred_element_type=jnp.float32)
    cntfe[...] += jnp.dot(afe, ones, preferred_element_type=jnp.float32)

    @pl.when(k == NK - 1)
    def _fin():
        of_ref[...] = accf[...] * (1.0 / jnp.maximum(cntf[...,  :1], 1.0))
        oe_ref[...] = acce[...] * (1.0 / jnp.maximum(cnte[...,  :1], 1.0))
        ofe_ref[...] = accfe[...] * (1.0 / jnp.maximum(cntfe[...,  :1], 1.0))


def _agg(a_ff, a_ee, a_fe, xf_flat, xe_flat):
    adj_spec = pl.BlockSpec((BI, BK), lambda i, k: (i, k))
    src_spec = pl.BlockSpec((BK, T * H), lambda i, k: (k, 0))
    out_spec = pl.BlockSpec((BI, T * H), lambda i, k: (i, 0))
    return pl.pallas_call(
        _agg_body,
        grid=(NI, NK),
        in_specs=[adj_spec, adj_spec, adj_spec, src_spec, src_spec],
        out_specs=[out_spec] * 3,
        out_shape=[jax.ShapeDtypeStruct((N, T * H), jnp.float32)] * 3,
        scratch_shapes=[pltpu.VMEM((BI, T * H), jnp.float32)] * 3
        + [pltpu.VMEM((BI, 128), jnp.float32)] * 3,
        compiler_params=pltpu.CompilerParams(
            dimension_semantics=("parallel", "arbitrary")),
    )(a_ff, a_ee, a_fe, xf_flat, xe_flat)


def _lin_body(xf_ref, xe_ref, agf_ref, age_ref, agfe_ref,
              wrf_ref, wnf_ref, bf_ref, wre_ref, wnee_ref, wnfe_ref, be_ref,
              of_ref, oe_ref):
    xdf = xf_ref[...].reshape(BI * T, H)
    xde = xe_ref[...].reshape(BI * T, H)
    aggf = agf_ref[...].reshape(BI * T, H)
    agge = age_ref[...].reshape(BI * T, H)
    aggfe = agfe_ref[...].reshape(BI * T, H)
    of = (jnp.dot(xdf, wrf_ref[...], preferred_element_type=jnp.float32)
          + jnp.dot(aggf, wnf_ref[...], preferred_element_type=jnp.float32)
          + bf_ref[...])
    oe = (jnp.dot(xde, wre_ref[...], preferred_element_type=jnp.float32)
          + jnp.dot(agge, wnee_ref[...], preferred_element_type=jnp.float32)
          + jnp.dot(aggfe, wnfe_ref[...], preferred_element_type=jnp.float32)
          + be_ref[...])
    of_ref[...] = jnp.maximum(of, 0.0).reshape(BI, T, H)
    oe_ref[...] = jnp.maximum(0.5 * oe, 0.0).reshape(BI, T, H)


def _lin(xf, xe, agf, age, agfe, wrf, wnf, bf, wre, wnee, wnfe, be):
    x_spec = pl.BlockSpec((BI, T, H), lambda i: (i, 0, 0))
    w_spec = pl.BlockSpec((H, H), lambda i: (0, 0))
    b_spec = pl.BlockSpec((1, H), lambda i: (0, 0))
    return pl.pallas_call(
        _lin_body,
        grid=(NI,),
        in_specs=[x_spec] * 5 + [w_spec, w_spec, b_spec,
                                 w_spec, w_spec, w_spec, b_spec],
        out_specs=[x_spec, x_spec],
        out_shape=[jax.ShapeDtypeStruct((N, T, H), jnp.float32)] * 2,
        compiler_params=pltpu.CompilerParams(
            dimension_semantics=("parallel",)),
    )(xf, xe, agf, age, agfe, wrf, wnf, bf, wre, wnee, wnfe, be)


def _sage_layer(a_ff, a_ee, a_fe, xf, xe,
                wrf, wnf, bf, wre, wnee, wnfe, be):
    agf, age, agfe = _agg(a_ff, a_ee, a_fe, xf.reshape(N, T * H),
                          xe.reshape(N, T * H))
    return _lin(xf, xe, agf.reshape(N, T, H), age.reshape(N, T, H),
                agfe.reshape(N, T, H), wrf, wnf, bf, wre, wnee, wnfe, be)


def _gru_body(x_ref, wih_ref, whh_ref, bih_ref, bhh_ref, ys_ref, hf_ref):
    x = x_ref[...].reshape(BN * T, H)
    gi = (jnp.dot(x, wih_ref[...], preferred_element_type=jnp.float32)
          + bih_ref[...]).reshape(BN, T, 3 * H)
    h = jnp.zeros((BN, H), jnp.float32)
    for t in range(T):
        gi_t = gi[:, t, :]
        gh = jnp.dot(h, whh_ref[...], preferred_element_type=jnp.float32) \
            + bhh_ref[...]
        r = jax.nn.sigmoid(gi_t[:, :H] + gh[:, :H])
        z = jax.nn.sigmoid(gi_t[:, H:2 * H] + gh[:, H:2 * H])
        n = jnp.tanh(gi_t[:, 2 * H:] + r * gh[:, 2 * H:])
        h = (1.0 - z) * n + z * h
        ys_ref[t] = h
    hf_ref[...] = h


def _gru(x, wihT, whhT, bih, bhh):
    return pl.pallas_call(
        _gru_body,
        grid=(NB,),
        in_specs=[pl.BlockSpec((BN, T, H), lambda i: (i, 0, 0)),
                  pl.BlockSpec((H, 3 * H), lambda i: (0, 0)),
                  pl.BlockSpec((H, 3 * H), lambda i: (0, 0)),
                  pl.BlockSpec((1, 3 * H), lambda i: (0, 0)),
                  pl.BlockSpec((1, 3 * H), lambda i: (0, 0))],
        out_specs=[pl.BlockSpec((T, BN, H), lambda i: (0, i, 0)),
                   pl.BlockSpec((BN, H), lambda i: (i, 0))],
        out_shape=[jax.ShapeDtypeStruct((T, N, H), jnp.float32),
                   jax.ShapeDtypeStruct((N, H), jnp.float32)],
        compiler_params=pltpu.CompilerParams(
            dimension_semantics=("parallel",)),
    )(x, wihT, whhT, bih, bhh)


def _dec_body(ys_ref, z_ref, wg_ref, wz_ref, db_ref, c1_ref, b1_ref,
              c2_ref, b2_ref, c3_ref, ls_ref, std_ref, mean_ref, out_ref,
              pad1, pad2):
    zW = jnp.dot(z_ref[...], wz_ref[...], preferred_element_type=jnp.float32)
    h = jnp.dot(ys_ref[...].reshape(T * BN, H), wg_ref[...],
                preferred_element_type=jnp.float32).reshape(T, BN, H)
    h = h + zW[None, :, :] + db_ref[...][None, :, :]

    pad1[...] = jnp.zeros_like(pad1)
    pad1[2:T + 2] = h
    acc1 = jnp.zeros((T * BN, C), jnp.float32)
    for k in range(K):
        xk = pad1[k:k + T].reshape(T * BN, H)
        acc1 += jnp.dot(xk, c1_ref[k], preferred_element_type=jnp.float32)
    h1 = jnp.maximum(acc1 + b1_ref[...], 0.0).reshape(T, BN, C)

    pad2[...] = jnp.zeros_like(pad2)
    pad2[2:T + 2] = h1
    acc2 = jnp.zeros((T * BN, C), jnp.float32)
    for k in range(K):
        xk = pad2[k:k + T].reshape(T * BN, C)
        acc2 += jnp.dot(xk, c2_ref[k], preferred_element_type=jnp.float32)
    h2 = jnp.maximum(acc2 + b2_ref[...], 0.0).reshape(T, BN, C)

    pad1[...] = jnp.zeros_like(pad1)
    pad1[2:T + 2] = h2
    tmp = jnp.zeros((T, BN, C), jnp.float32)
    for k in range(K):
        tmp += pad1[k:k + T] * c3_ref[0, k][None, None, :]
    acc3 = jnp.sum(tmp, axis=-1) + ls_ref[1]
    scale = std_ref[...] * jnp.exp(ls_ref[0])
    out_ref[...] = acc3 * scale + mean_ref[...]


def _decoder(ys, z, wgT, wzT, db, c1T, b1, c2T, b2, c3T, ls2, std, mean):
    return pl.pallas_call(
        _dec_body,
        grid=(NB,),
        in_specs=[pl.BlockSpec((T, BN, H), lambda i: (0, i, 0)),
                  pl.BlockSpec((BN, H), lambda i: (i, 0)),
                  pl.BlockSpec((H, H), lambda i: (0, 0)),
                  pl.BlockSpec((H, H), lambda i: (0, 0)),
                  pl.BlockSpec((1, H), lambda i: (0, 0)),
                  pl.BlockSpec((K, H, C), lambda i: (0, 0, 0)),
                  pl.BlockSpec((1, C), lambda i: (0, 0)),
                  pl.BlockSpec((K, C, C), lambda i: (0, 0, 0)),
                  pl.BlockSpec((1, C), lambda i: (0, 0)),
                  pl.BlockSpec((1, K, C), lambda i: (0, 0, 0)),
                  pl.BlockSpec(memory_space=pltpu.SMEM),
                  pl.BlockSpec((1, BN), lambda i: (0, i)),
                  pl.BlockSpec((1, BN), lambda i: (0, i))],
        out_specs=pl.BlockSpec((T, BN), lambda i: (0, i)),
        out_shape=jax.ShapeDtypeStruct((T, N), jnp.float32),
        scratch_shapes=[pltpu.VMEM((T + 4, BN, C), jnp.float32)] * 2,
        compiler_params=pltpu.CompilerParams(
            dimension_semantics=("arbitrary",)),
    )(ys, z, wgT, wzT, db, c1T, b1, c2T, b2, c3T, ls2, std, mean)


def _head_body(ysf_ref, yse_ref, zf_ref, ze_ref, wbar_ref,
               gw1_ref, gb1_ref, gw2_ref, gb2_ref,
               pf_ref, pe_ref, gs_ref, zsum):
    i = pl.program_id(0)

    @pl.when(i == 0)
    def _init():
        zsum[...] = jnp.zeros_like(zsum)

    pf = jnp.zeros((BN, H), jnp.float32)
    pe = jnp.zeros((BN, H), jnp.float32)
    for t in range(T):
        pf += ysf_ref[t] * wbar_ref[0, t]
        pe += yse_ref[t] * wbar_ref[1, t]
    pf_ref[...] = pf
    pe_ref[...] = pe
    zsum[...] += (jnp.sum(zf_ref[...], axis=0, keepdims=True)
                  + jnp.sum(ze_ref[...], axis=0, keepdims=True))

    @pl.when(i == NB - 1)
    def _fin():
        g = zsum[...] * (0.5 / N)
        t1 = jnp.maximum(
            jnp.dot(g, gw1_ref[...], preferred_element_type=jnp.float32)
            + gb1_ref[...], 0.0)
        gs_ref[...] = jnp.dot(t1, gw2_ref[...],
                              preferred_element_type=jnp.float32) + gb2_ref[...]


def _head(ysf, yse, zf, ze, wbar, gw1T, gb1, gw2T, gb2):
    return pl.pallas_call(
        _head_body,
        grid=(NB,),
        in_specs=[pl.BlockSpec((T, BN, H), lambda i: (0, i, 0)),
                  pl.BlockSpec((T, BN, H), lambda i: (0, i, 0)),
                  pl.BlockSpec((BN, H), lambda i: (i, 0)),
                  pl.BlockSpec((BN, H), lambda i: (i, 0)),
                  pl.BlockSpec(memory_space=pltpu.SMEM),
                  pl.BlockSpec((H, H), lambda i: (0, 0)),
                  pl.BlockSpec((1, H), lambda i: (0, 0)),
                  pl.BlockSpec((H, H), lambda i: (0, 0)),
                  pl.BlockSpec((1, H), lambda i: (0, 0))],
        out_specs=[pl.BlockSpec((BN, H), lambda i: (i, 0)),
                   pl.BlockSpec((BN, H), lambda i: (i, 0)),
                   pl.BlockSpec((1, H), lambda i: (0, 0))],
        out_shape=[jax.ShapeDtypeStruct((N, H), jnp.float32),
                   jax.ShapeDtypeStruct((N, H), jnp.float32),
                   jax.ShapeDtypeStruct((1, H), jnp.float32)],
        scratch_shapes=[pltpu.VMEM((1, H), jnp.float32)],
        compiler_params=pltpu.CompilerParams(
            dimension_semantics=("arbitrary",)),
    )(ysf, yse, zf, ze, wbar, gw1T, gb1, gw2T, gb2)


SLAB_ROWS = 32 * (N // 16)
TRASH = SLAB_ROWS
ECHUNK = 4096
NCHUNK = E // ECHUNK


SLAB_ELEMS = 32 * N
TRASH_BASE = SLAB_ELEMS


def _adj_body(sff, dff, see, dee, sfe, dfe, adj, s_buf, d_buf, slab):
    c = lax.axis_index("c")
    s = lax.axis_index("s")
    w = c * 16 + s
    lanes = lax.iota(jnp.int32, 16)
    zvec = jnp.zeros((16,), jnp.float32)

    for rel, (src_h, dst_h) in enumerate(((sff, dff), (see, dee), (sfe, dfe))):
        for sub in range(2):
            slab_id = w * 2 + sub
            r0 = slab_id * 32

            def zbody(i, _):
                for k in range(8):
                    slab[pl.ds((i * 8 + k) * 16, 16)] = zvec
                return 0
            lax.fori_loop(0, (SLAB_ELEMS + 16) // 128, zbody, 0)

            def chunk_body(cidx, _):
                pltpu.sync_copy(src_h.at[pl.ds(cidx * ECHUNK, ECHUNK)], s_buf)
                pltpu.sync_copy(dst_h.at[pl.ds(cidx * ECHUNK, ECHUNK)], d_buf)

                def vec_body(v, _):
                    for u in range(4):
                        off = (v * 4 + u) * 16
                        sv = s_buf[pl.ds(off, 16)]
                        dv = d_buf[pl.ds(off, 16)]
                        loc = dv - r0
                        valid = (loc >= 0) & (loc < 32)
                        flat = jnp.where(valid, loc * N + sv,
                                         TRASH_BASE + lanes)
                        m, last = plsc.scan_count(flat)
                        plsc.addupdate_scatter(
                            slab, [flat], m.astype(jnp.float32), mask=last)
                    return 0
                lax.fori_loop(0, ECHUNK // 64, vec_body, 0)
                return 0
            lax.fori_loop(0, NCHUNK, chunk_body, 0)

            hbm_off = rel * N * N + slab_id * SLAB_ELEMS
            pltpu.sync_copy(slab.at[pl.ds(0, SLAB_ELEMS)],
                            adj.at[pl.ds(hbm_off, SLAB_ELEMS)])


def _build_adj_sc(ei_ff, ei_ee, ei_fe):
    mesh = plsc.VectorSubcoreMesh(core_axis_name="c", subcore_axis_name="s")
    f = pl.kernel(
        _adj_body,
        mesh=mesh,
        out_type=jax.ShapeDtypeStruct((3 * N * N,), jnp.float32),
        scratch_types=[
            pltpu.VMEM((ECHUNK,), jnp.int32),
            pltpu.VMEM((ECHUNK,), jnp.int32),
            pltpu.VMEM((SLAB_ELEMS + 16,), jnp.float32),
        ],
        compiler_params=pltpu.CompilerParams(needs_layout_passes=False,
                                             use_tc_tiling_on_sc=False),
    )
    i32 = jnp.int32
    adj = f(ei_ff[0].astype(i32), ei_ff[1].astype(i32),
            ei_ee[0].astype(i32), ei_ee[1].astype(i32),
            ei_fe[0].astype(i32), ei_fe[1].astype(i32))
    return adj.reshape(3, N, N)


def kernel(encoded_fmri, encoded_eeg, mean_fmri, std_fmri, mean_eeg, std_eeg,
           sage_Wr, sage_Wn, sage_b, gru_Wih, gru_Whh, gru_bih, gru_bhh,
           tproj_W, dec_in_W, dec_in_b, c1W, c1b, c2W, c2b, c3W, c3b,
           log_scale, gW1, gb1, gW2, gb2, ei_ff, ei_ee, ei_fe):
    adj3 = _build_adj_sc(ei_ff, ei_ee, ei_fe)
    a_ff, a_ee, a_fe = adj3[0], adj3[1], adj3[2]

    xf, xe = encoded_fmri, encoded_eeg
    for l in range(L):
        xf, xe = _sage_layer(
            a_ff, a_ee, a_fe, xf, xe,
            sage_Wr[l, 0].T, sage_Wn[l, 0].T, sage_b[l, 0].reshape(1, H),
            (sage_Wr[l, 1] + sage_Wr[l, 2]).T, sage_Wn[l, 1].T,
            sage_Wn[l, 2].T, (sage_b[l, 1] + sage_b[l, 2]).reshape(1, H))

    ysf, zf = _gru(xf, gru_Wih[0].T, gru_Whh[0].T,
                   gru_bih[0].reshape(1, 3 * H), gru_bhh[0].reshape(1, 3 * H))
    yse, ze = _gru(xe, gru_Wih[1].T, gru_Whh[1].T,
                   gru_bih[1].reshape(1, 3 * H), gru_bhh[1].reshape(1, 3 * H))

    rf = _decoder(ysf, zf, dec_in_W[0, :, :H].T, dec_in_W[0, :, H:].T,
                  dec_in_b[0].reshape(1, H), c1W[0].transpose(2, 1, 0),
                  c1b[0].reshape(1, C), c2W[0].transpose(2, 1, 0),
                  c2b[0].reshape(1, C), c3W[0].transpose(0, 2, 1),
                  jnp.stack([log_scale[0], c3b[0, 0]]),
                  std_fmri.reshape(1, N), mean_fmri.reshape(1, N))
    re = _decoder(yse, ze, dec_in_W[1, :, :H].T, dec_in_W[1, :, H:].T,
                  dec_in_b[1].reshape(1, H), c1W[1].transpose(2, 1, 0),
                  c1b[1].reshape(1, C), c2W[1].transpose(2, 1, 0),
                  c2b[1].reshape(1, C), c3W[1].transpose(0, 2, 1),
                  jnp.stack([log_scale[1], c3b[1, 0]]),
                  std_eeg.reshape(1, N), mean_eeg.reshape(1, N))

    wbar = tproj_W.mean(axis=1)
    pf, pe, gs = _head(ysf, yse, zf, ze, wbar, gW1.T, gb1.reshape(1, H),
                       gW2.T, gb2.reshape(1, H))

    return jnp.concatenate([rf.T.ravel(), re.T.ravel(), zf.ravel(),
                            ze.ravel(), pf.ravel(), pe.ravel(), gs.ravel()])
